# Initial kernel scaffold; baseline (speedup 1.0000x reference)
import jax
import jax.numpy as jnp
from jax.experimental import pallas as pl


def kernel(target_words, context_words, negative_words, input_emb, output_emb):
    raise NotImplementedError("write your pallas kernel here")


# speedup vs baseline: 3.1415x; 3.1415x over previous
import functools

import jax
import jax.numpy as jnp
from jax import lax
from jax.experimental import pallas as pl
from jax.experimental.pallas import tpu as pltpu
from jax.experimental.pallas import tpu_sc as plsc

VOCAB = 100000
DIM = 128
BATCH = 16384
NEG = 5

_info = plsc.get_sparse_core_info()
_NC, _NS, _L = _info.num_cores, _info.num_subcores, _info.num_lanes
_NW = _NC * _NS
_BPW = BATCH // _NW
_CHUNK = 64
_NCHUNKS = _BPW // _CHUNK
_NGROUPS = _CHUNK // _L

_mesh = plsc.VectorSubcoreMesh(core_axis_name="c", subcore_axis_name="s")


@functools.partial(
    pl.kernel,
    mesh=_mesh,
    compiler_params=pltpu.CompilerParams(needs_layout_passes=False),
    out_type=(
        jax.ShapeDtypeStruct((BATCH,), jnp.float32),
        jax.ShapeDtypeStruct((_NW, NEG, _BPW), jnp.float32),
    ),
    scratch_types=[
        pltpu.VMEM((_BPW,), jnp.int32),
        pltpu.VMEM((_BPW,), jnp.int32),
        pltpu.VMEM((_BPW * NEG,), jnp.int32),
        pltpu.VMEM((_CHUNK, DIM), jnp.float32),
        pltpu.VMEM((_CHUNK, DIM), jnp.float32),
        pltpu.VMEM((_CHUNK, DIM), jnp.float32),
        pltpu.VMEM((_CHUNK, DIM), jnp.float32),
        pltpu.VMEM((_CHUNK * NEG, DIM), jnp.float32),
        pltpu.VMEM((_CHUNK * NEG, DIM), jnp.float32),
        pltpu.VMEM((_BPW,), jnp.float32),
        pltpu.VMEM((NEG, _BPW), jnp.float32),
        pltpu.SemaphoreType.DMA,
        pltpu.SemaphoreType.DMA,
    ],
)
def _sc_scores(tgt_idx_hbm, ctx_idx_hbm, neg_idx_hbm, in_emb_hbm, out_emb_hbm,
               pos_out_hbm, neg_out_hbm,
               idx_t_v, idx_c_v, idx_n_v,
               tgt_v0, tgt_v1, ctx_v0, ctx_v1, neg_v0, neg_v1,
               pos_s_v, neg_s_v, sem0, sem1):
    wid = lax.axis_index("s") * _NC + lax.axis_index("c")
    base = wid * _BPW

    pltpu.sync_copy(tgt_idx_hbm.at[pl.ds(base, _BPW)], idx_t_v)
    pltpu.sync_copy(ctx_idx_hbm.at[pl.ds(base, _BPW)], idx_c_v)
    pltpu.sync_copy(neg_idx_hbm.at[pl.ds(base * NEG, _BPW * NEG)], idx_n_v)

    bufs = ((tgt_v0, ctx_v0, neg_v0, sem0), (tgt_v1, ctx_v1, neg_v1, sem1))

    def fire(c):
        t_b, c_b, n_b, s_b = bufs[c % 2]
        o = c * _CHUNK
        return (
            pltpu.async_copy(
                in_emb_hbm.at[idx_t_v.at[pl.ds(o, _CHUNK)]], t_b, s_b),
            pltpu.async_copy(
                out_emb_hbm.at[idx_c_v.at[pl.ds(o, _CHUNK)]], c_b, s_b),
            pltpu.async_copy(
                out_emb_hbm.at[idx_n_v.at[pl.ds(o * NEG, _CHUNK * NEG)]],
                n_b, s_b),
        )

    lanes = lax.iota(jnp.int32, _L)
    zero = jnp.zeros((_L,), jnp.float32)

    pending = fire(0)
    for c in range(_NCHUNKS):
        nxt = fire(c + 1) if c + 1 < _NCHUNKS else None
        for cp in pending:
            cp.wait()
        pending = nxt

        tgt_v, ctx_v, neg_v, _ = bufs[c % 2]
        off = c * _CHUNK

        def group_body(g, _, tgt_v=tgt_v, ctx_v=ctx_v, neg_v=neg_v, off=off):
            def row_body(i, res):
                r = g * _L + i
                accs = [zero for _ in range(1 + NEG)]
                for q in range(DIM // _L):
                    sl = pl.ds(q * _L, _L)
                    t = tgt_v[r, sl]
                    accs[0] = accs[0] + t * ctx_v[r, sl]
                    for k in range(NEG):
                        accs[1 + k] = accs[1 + k] + t * neg_v[r * NEG + k, sl]
                m = lanes == i
                return tuple(
                    jnp.where(m, jnp.sum(a), res[d]) for d, a in enumerate(accs)
                )

            res = lax.fori_loop(0, _L, row_body, (zero,) * (1 + NEG))
            pos_s_v[pl.ds(off + g * _L, _L)] = res[0]
            for k in range(NEG):
                neg_s_v[k, pl.ds(off + g * _L, _L)] = res[1 + k]
            return 0

        lax.fori_loop(0, _NGROUPS, group_body, 0)

    pltpu.sync_copy(pos_s_v, pos_out_hbm.at[pl.ds(base, _BPW)])
    pltpu.sync_copy(neg_s_v, neg_out_hbm.at[wid])


def _loss_body(pos_ref, neg_ref, pos_loss_ref, neg_loss_ref):
    p = pos_ref[...]
    lsp = jnp.minimum(p, 0.0) - jnp.log1p(jnp.exp(-jnp.abs(p)))
    pos_loss_ref[0, 0] = -jnp.sum(lsp) / float(BATCH)
    x = -neg_ref[...]
    lsn = jnp.minimum(x, 0.0) - jnp.log1p(jnp.exp(-jnp.abs(x)))
    neg_loss_ref[0, 0] = -jnp.sum(lsn) / float(BATCH * NEG)


_loss_call = pl.pallas_call(
    _loss_body,
    out_shape=(
        jax.ShapeDtypeStruct((1, 1), jnp.float32),
        jax.ShapeDtypeStruct((1, 1), jnp.float32),
    ),
    out_specs=(
        pl.BlockSpec(memory_space=pltpu.SMEM),
        pl.BlockSpec(memory_space=pltpu.SMEM),
    ),
)


def kernel(target_words, context_words, negative_words, input_emb, output_emb):
    tw = target_words.astype(jnp.int32)
    cw = context_words.astype(jnp.int32)
    nw = negative_words.astype(jnp.int32).reshape(-1)
    pos_scores, neg_scores = _sc_scores(tw, cw, nw, input_emb, output_emb)
    pos2d = pos_scores.reshape(BATCH // DIM, DIM)
    neg2d = neg_scores.reshape(_NW * NEG, _BPW)
    pos_loss, neg_loss = _loss_call(pos2d, neg2d)
    return (pos_loss[0, 0], neg_loss[0, 0])


# speedup vs baseline: 3.7396x; 1.1904x over previous
import functools

import jax
import jax.numpy as jnp
from jax import lax
from jax.experimental import pallas as pl
from jax.experimental.pallas import tpu as pltpu
from jax.experimental.pallas import tpu_sc as plsc

VOCAB = 100000
DIM = 128
BATCH = 16384
NEG = 5

_info = plsc.get_sparse_core_info()
_NC, _NS, _L = _info.num_cores, _info.num_subcores, _info.num_lanes
_NW = _NC * _NS
_BPW = BATCH // _NW
_CHUNK = 64
_NCHUNKS = _BPW // _CHUNK
_NGROUPS = _CHUNK // _L

_mesh = plsc.VectorSubcoreMesh(core_axis_name="c", subcore_axis_name="s")


@functools.partial(
    pl.kernel,
    mesh=_mesh,
    compiler_params=pltpu.CompilerParams(needs_layout_passes=False),
    out_type=(
        jax.ShapeDtypeStruct((1, BATCH), jnp.float32),
        jax.ShapeDtypeStruct((NEG, BATCH), jnp.float32),
    ),
    scratch_types=[
        pltpu.VMEM((_BPW,), jnp.int32),
        pltpu.VMEM((_BPW,), jnp.int32),
        pltpu.VMEM((NEG, _BPW), jnp.int32),
        pltpu.VMEM((_CHUNK, DIM), jnp.float32),
        pltpu.VMEM((_CHUNK, DIM), jnp.float32),
        pltpu.VMEM((_CHUNK, DIM), jnp.float32),
        pltpu.VMEM((_CHUNK, DIM), jnp.float32),
        pltpu.VMEM((_CHUNK * NEG, DIM), jnp.float32),
        pltpu.VMEM((_CHUNK * NEG, DIM), jnp.float32),
        pltpu.VMEM((1, _BPW), jnp.float32),
        pltpu.VMEM((NEG, _BPW), jnp.float32),
        pltpu.SemaphoreType.DMA,
        pltpu.SemaphoreType.DMA,
    ],
)
def _sc_scores(tgt_idx_hbm, ctx_idx_hbm, neg_idx_hbm, in_emb_hbm, out_emb_hbm,
               pos_out_hbm, neg_out_hbm,
               idx_t_v, idx_c_v, idx_n_v,
               tgt_v0, tgt_v1, ctx_v0, ctx_v1, neg_v0, neg_v1,
               pos_s_v, neg_s_v, sem0, sem1):
    wid = lax.axis_index("s") * _NC + lax.axis_index("c")
    base = wid * _BPW

    pltpu.sync_copy(tgt_idx_hbm.at[pl.ds(base, _BPW)], idx_t_v)
    pltpu.sync_copy(ctx_idx_hbm.at[pl.ds(base, _BPW)], idx_c_v)
    pltpu.sync_copy(neg_idx_hbm.at[:, pl.ds(base, _BPW)], idx_n_v)

    bufs = ((tgt_v0, ctx_v0, neg_v0, sem0), (tgt_v1, ctx_v1, neg_v1, sem1))

    def fire(c):
        t_b, c_b, n_b, s_b = bufs[c % 2]
        o = c * _CHUNK
        return (
            pltpu.async_copy(
                in_emb_hbm.at[idx_t_v.at[pl.ds(o, _CHUNK)]], t_b, s_b),
            pltpu.async_copy(
                out_emb_hbm.at[idx_c_v.at[pl.ds(o, _CHUNK)]], c_b, s_b),
        ) + tuple(
            pltpu.async_copy(
                out_emb_hbm.at[idx_n_v.at[k, pl.ds(o, _CHUNK)]],
                n_b.at[pl.ds(k * _CHUNK, _CHUNK), :], s_b)
            for k in range(NEG)
        )

    lanes = lax.iota(jnp.int32, _L)
    zero = jnp.zeros((_L,), jnp.float32)

    pending = fire(0)
    for c in range(_NCHUNKS):
        nxt = fire(c + 1) if c + 1 < _NCHUNKS else None
        for cp in pending:
            cp.wait()
        pending = nxt

        tgt_v, ctx_v, neg_v, _ = bufs[c % 2]
        off = c * _CHUNK

        def group_body(g, _, tgt_v=tgt_v, ctx_v=ctx_v, neg_v=neg_v, off=off):
            def row_body(i, res):
                r = g * _L + i
                accs = [zero for _ in range(1 + NEG)]
                for q in range(DIM // _L):
                    sl = pl.ds(q * _L, _L)
                    t = tgt_v[r, sl]
                    accs[0] = accs[0] + t * ctx_v[r, sl]
                    for k in range(NEG):
                        accs[1 + k] = accs[1 + k] + t * neg_v[k * _CHUNK + r, sl]
                m = lanes == i
                return tuple(
                    jnp.where(m, jnp.sum(a), res[d]) for d, a in enumerate(accs)
                )

            res = lax.fori_loop(0, _L, row_body, (zero,) * (1 + NEG))
            v = off + g * _L
            pos_s_v[0, pl.ds(v, _L)] = res[0]
            for k in range(NEG):
                neg_s_v[k, pl.ds(v, _L)] = res[1 + k]
            return 0

        lax.fori_loop(0, _NGROUPS, group_body, 0)

    pltpu.sync_copy(pos_s_v, pos_out_hbm.at[:, pl.ds(base, _BPW)])
    pltpu.sync_copy(neg_s_v, neg_out_hbm.at[:, pl.ds(base, _BPW)])


def _loss_body(pos_ref, neg_ref, pos_loss_ref, neg_loss_ref):
    p = pos_ref[...]
    lsp = jnp.minimum(p, 0.0) - jnp.log1p(jnp.exp(-jnp.abs(p)))
    pos_loss_ref[0, 0] = -jnp.sum(lsp) / float(BATCH)
    x = -neg_ref[...]
    lsn = jnp.minimum(x, 0.0) - jnp.log1p(jnp.exp(-jnp.abs(x)))
    neg_loss_ref[0, 0] = -jnp.sum(lsn) / float(BATCH * NEG)


_loss_call = pl.pallas_call(
    _loss_body,
    out_shape=(
        jax.ShapeDtypeStruct((1, 1), jnp.float32),
        jax.ShapeDtypeStruct((1, 1), jnp.float32),
    ),
    out_specs=(
        pl.BlockSpec(memory_space=pltpu.SMEM),
        pl.BlockSpec(memory_space=pltpu.SMEM),
    ),
)


def kernel(target_words, context_words, negative_words, input_emb, output_emb):
    tw = target_words.astype(jnp.int32)
    cw = context_words.astype(jnp.int32)
    nw = negative_words.astype(jnp.int32).T
    pos_scores, neg_scores = _sc_scores(tw, cw, nw, input_emb, output_emb)
    pos_loss, neg_loss = _loss_call(pos_scores, neg_scores)
    return (pos_loss[0, 0], neg_loss[0, 0])


# speedup vs baseline: 3.9647x; 1.0602x over previous
import functools

import jax
import jax.numpy as jnp
from jax import lax
from jax.experimental import pallas as pl
from jax.experimental.pallas import tpu as pltpu
from jax.experimental.pallas import tpu_sc as plsc

VOCAB = 100000
DIM = 128
BATCH = 16384
NEG = 5

_info = plsc.get_sparse_core_info()
_NC, _NS, _L = _info.num_cores, _info.num_subcores, _info.num_lanes
_NW = _NC * _NS
_BPW = BATCH // _NW
_CHUNK = 64
_NCHUNKS = _BPW // _CHUNK
_NGROUPS = _CHUNK // _L

_mesh = plsc.VectorSubcoreMesh(core_axis_name="c", subcore_axis_name="s")


@functools.partial(
    pl.kernel,
    mesh=_mesh,
    compiler_params=pltpu.CompilerParams(needs_layout_passes=False),
    out_type=(
        jax.ShapeDtypeStruct((1, BATCH), jnp.float32),
        jax.ShapeDtypeStruct((NEG, BATCH), jnp.float32),
    ),
    scratch_types=[
        pltpu.VMEM((_BPW,), jnp.int32),
        pltpu.VMEM((_BPW,), jnp.int32),
        pltpu.VMEM((NEG, _BPW), jnp.int32),
        pltpu.VMEM((_CHUNK, DIM), jnp.float32),
        pltpu.VMEM((_CHUNK, DIM), jnp.float32),
        pltpu.VMEM((_CHUNK, DIM), jnp.float32),
        pltpu.VMEM((_CHUNK, DIM), jnp.float32),
        pltpu.VMEM((_CHUNK * NEG, DIM), jnp.float32),
        pltpu.VMEM((_CHUNK * NEG, DIM), jnp.float32),
        pltpu.VMEM((1, _BPW), jnp.float32),
        pltpu.VMEM((NEG, _BPW), jnp.float32),
        pltpu.SemaphoreType.DMA,
        pltpu.SemaphoreType.DMA,
    ],
)
def _sc_scores(tgt_idx_hbm, ctx_idx_hbm, neg_idx_hbm, in_emb_hbm, out_emb_hbm,
               pos_out_hbm, neg_out_hbm,
               idx_t_v, idx_c_v, idx_n_v,
               tgt_v0, tgt_v1, ctx_v0, ctx_v1, neg_v0, neg_v1,
               pos_s_v, neg_s_v, sem0, sem1):
    wid = lax.axis_index("s") * _NC + lax.axis_index("c")
    base = wid * _BPW

    pltpu.sync_copy(tgt_idx_hbm.at[pl.ds(base, _BPW)], idx_t_v)
    pltpu.sync_copy(ctx_idx_hbm.at[pl.ds(base, _BPW)], idx_c_v)
    pltpu.sync_copy(neg_idx_hbm.at[:, pl.ds(base, _BPW)], idx_n_v)

    bufs = ((tgt_v0, ctx_v0, neg_v0, sem0), (tgt_v1, ctx_v1, neg_v1, sem1))

    def fire(par, o):
        t_b, c_b, n_b, s_b = bufs[par]
        pltpu.async_copy(in_emb_hbm.at[idx_t_v.at[pl.ds(o, _CHUNK)]], t_b, s_b)
        pltpu.async_copy(out_emb_hbm.at[idx_c_v.at[pl.ds(o, _CHUNK)]], c_b, s_b)
        for k in range(NEG):
            pltpu.async_copy(
                out_emb_hbm.at[idx_n_v.at[k, pl.ds(o, _CHUNK)]],
                n_b.at[pl.ds(k * _CHUNK, _CHUNK), :], s_b)

    def wait(par):
        t_b, c_b, n_b, s_b = bufs[par]
        pltpu.make_async_copy(
            in_emb_hbm.at[idx_t_v.at[pl.ds(0, _CHUNK)]], t_b, s_b).wait()
        pltpu.make_async_copy(
            out_emb_hbm.at[idx_c_v.at[pl.ds(0, _CHUNK)]], c_b, s_b).wait()
        for k in range(NEG):
            pltpu.make_async_copy(
                out_emb_hbm.at[idx_n_v.at[k, pl.ds(0, _CHUNK)]],
                n_b.at[pl.ds(k * _CHUNK, _CHUNK), :], s_b).wait()

    lanes = lax.iota(jnp.int32, _L)
    zero = jnp.zeros((_L,), jnp.float32)

    def compute(par, off):
        tgt_v, ctx_v, neg_v, _ = bufs[par]

        def group_body(g, _):
            def row_body(i, res):
                r = g * _L + i
                accs = [zero for _ in range(1 + NEG)]
                for q in range(DIM // _L):
                    sl = pl.ds(q * _L, _L)
                    t = tgt_v[r, sl]
                    accs[0] = accs[0] + t * ctx_v[r, sl]
                    for k in range(NEG):
                        accs[1 + k] = accs[1 + k] + t * neg_v[k * _CHUNK + r, sl]
                m = lanes == i
                return tuple(
                    jnp.where(m, jnp.sum(a), res[d]) for d, a in enumerate(accs)
                )

            res = lax.fori_loop(0, _L, row_body, (zero,) * (1 + NEG))
            v = off + g * _L
            pos_s_v[0, pl.ds(v, _L)] = res[0]
            for k in range(NEG):
                neg_s_v[k, pl.ds(v, _L)] = res[1 + k]
            return 0

        lax.fori_loop(0, _NGROUPS, group_body, 0)

    fire(0, 0)

    def pair_body(h, _):
        o0 = 2 * h * _CHUNK
        fire(1, o0 + _CHUNK)
        wait(0)
        compute(0, o0)

        @pl.when(h < _NCHUNKS // 2 - 1)
        def _():
            fire(0, o0 + 2 * _CHUNK)

        wait(1)
        compute(1, o0 + _CHUNK)
        return 0

    lax.fori_loop(0, _NCHUNKS // 2, pair_body, 0)

    pltpu.sync_copy(pos_s_v, pos_out_hbm.at[:, pl.ds(base, _BPW)])
    pltpu.sync_copy(neg_s_v, neg_out_hbm.at[:, pl.ds(base, _BPW)])


def _loss_body(pos_ref, neg_ref, pos_loss_ref, neg_loss_ref):
    p = pos_ref[...]
    lsp = jnp.minimum(p, 0.0) - jnp.log1p(jnp.exp(-jnp.abs(p)))
    pos_loss_ref[0, 0] = -jnp.sum(lsp) / float(BATCH)
    x = -neg_ref[...]
    lsn = jnp.minimum(x, 0.0) - jnp.log1p(jnp.exp(-jnp.abs(x)))
    neg_loss_ref[0, 0] = -jnp.sum(lsn) / float(BATCH * NEG)


_loss_call = pl.pallas_call(
    _loss_body,
    out_shape=(
        jax.ShapeDtypeStruct((1, 1), jnp.float32),
        jax.ShapeDtypeStruct((1, 1), jnp.float32),
    ),
    out_specs=(
        pl.BlockSpec(memory_space=pltpu.SMEM),
        pl.BlockSpec(memory_space=pltpu.SMEM),
    ),
)


def kernel(target_words, context_words, negative_words, input_emb, output_emb):
    tw = target_words.astype(jnp.int32)
    cw = context_words.astype(jnp.int32)
    nw = negative_words.astype(jnp.int32).T
    pos_scores, neg_scores = _sc_scores(tw, cw, nw, input_emb, output_emb)
    pos_loss, neg_loss = _loss_call(pos_scores, neg_scores)
    return (pos_loss[0, 0], neg_loss[0, 0])


# speedup vs baseline: 4.0151x; 1.0127x over previous
import functools

import jax
import jax.numpy as jnp
from jax import lax
from jax.experimental import pallas as pl
from jax.experimental.pallas import tpu as pltpu
from jax.experimental.pallas import tpu_sc as plsc

VOCAB = 100000
DIM = 128
BATCH = 16384
NEG = 5

_info = plsc.get_sparse_core_info()
_NC, _NS, _L = _info.num_cores, _info.num_subcores, _info.num_lanes
_NW = _NC * _NS
_BPW = BATCH // _NW
_CHUNK = 64
_NCHUNKS = _BPW // _CHUNK
_NGROUPS = _CHUNK // _L

_mesh = plsc.VectorSubcoreMesh(core_axis_name="c", subcore_axis_name="s")


@functools.partial(
    pl.kernel,
    mesh=_mesh,
    compiler_params=pltpu.CompilerParams(needs_layout_passes=False),
    out_type=(
        jax.ShapeDtypeStruct((1, BATCH), jnp.float32),
        jax.ShapeDtypeStruct((NEG, BATCH), jnp.float32),
    ),
    scratch_types=[
        pltpu.VMEM((_BPW,), jnp.int32),
        pltpu.VMEM((_BPW,), jnp.int32),
        pltpu.VMEM((NEG, _BPW), jnp.int32),
        pltpu.VMEM((2 * _CHUNK, DIM), jnp.float32),
        pltpu.VMEM((2 * _CHUNK, DIM), jnp.float32),
        pltpu.VMEM((2 * _CHUNK * NEG, DIM), jnp.float32),
        pltpu.VMEM((1, _BPW), jnp.float32),
        pltpu.VMEM((NEG, _BPW), jnp.float32),
        pltpu.SemaphoreType.DMA((2,)),
    ],
)
def _sc_scores(tgt_idx_hbm, ctx_idx_hbm, neg_idx_hbm, in_emb_hbm, out_emb_hbm,
               pos_out_hbm, neg_out_hbm,
               idx_t_v, idx_c_v, idx_n_v,
               tgt_v, ctx_v, neg_v,
               pos_s_v, neg_s_v, sems):
    wid = lax.axis_index("s") * _NC + lax.axis_index("c")
    base = wid * _BPW

    pltpu.sync_copy(tgt_idx_hbm.at[pl.ds(base, _BPW)], idx_t_v)
    pltpu.sync_copy(ctx_idx_hbm.at[pl.ds(base, _BPW)], idx_c_v)
    pltpu.sync_copy(neg_idx_hbm.at[:, pl.ds(base, _BPW)], idx_n_v)

    def fire(c):
        par = lax.rem(c, 2)
        po = par * _CHUNK
        o = c * _CHUNK
        s_b = sems.at[par]
        pltpu.async_copy(
            in_emb_hbm.at[idx_t_v.at[pl.ds(o, _CHUNK)]],
            tgt_v.at[pl.ds(po, _CHUNK), :], s_b)
        pltpu.async_copy(
            out_emb_hbm.at[idx_c_v.at[pl.ds(o, _CHUNK)]],
            ctx_v.at[pl.ds(po, _CHUNK), :], s_b)
        for k in range(NEG):
            pltpu.async_copy(
                out_emb_hbm.at[idx_n_v.at[k, pl.ds(o, _CHUNK)]],
                neg_v.at[pl.ds(po * NEG + k * _CHUNK, _CHUNK), :], s_b)

    def wait(c):
        par = lax.rem(c, 2)
        po = par * _CHUNK
        s_b = sems.at[par]
        pltpu.make_async_copy(
            in_emb_hbm.at[idx_t_v.at[pl.ds(0, _CHUNK)]],
            tgt_v.at[pl.ds(po, _CHUNK), :], s_b).wait()
        pltpu.make_async_copy(
            out_emb_hbm.at[idx_c_v.at[pl.ds(0, _CHUNK)]],
            ctx_v.at[pl.ds(po, _CHUNK), :], s_b).wait()
        for k in range(NEG):
            pltpu.make_async_copy(
                out_emb_hbm.at[idx_n_v.at[k, pl.ds(0, _CHUNK)]],
                neg_v.at[pl.ds(po * NEG + k * _CHUNK, _CHUNK), :], s_b).wait()

    lanes = lax.iota(jnp.int32, _L)
    zero = jnp.zeros((_L,), jnp.float32)

    def compute(c):
        par = lax.rem(c, 2)
        po = par * _CHUNK
        off = c * _CHUNK

        def group_body(g, _):
            def row_body(i, res):
                r = g * _L + i
                accs = [zero for _ in range(1 + NEG)]
                for q in range(DIM // _L):
                    sl = pl.ds(q * _L, _L)
                    t = tgt_v[po + r, sl]
                    accs[0] = accs[0] + t * ctx_v[po + r, sl]
                    for k in range(NEG):
                        accs[1 + k] = accs[1 + k] + t * neg_v[
                            po * NEG + k * _CHUNK + r, sl]
                m = lanes == i
                return tuple(
                    jnp.where(m, jnp.sum(a), res[d]) for d, a in enumerate(accs)
                )

            res = lax.fori_loop(0, _L, row_body, (zero,) * (1 + NEG))
            v = off + g * _L
            pos_s_v[0, pl.ds(v, _L)] = res[0]
            for k in range(NEG):
                neg_s_v[k, pl.ds(v, _L)] = res[1 + k]
            return 0

        lax.fori_loop(0, _NGROUPS, group_body, 0)

    fire(0)

    def chunk_body(c, _):
        @pl.when(c + 1 < _NCHUNKS)
        def _():
            fire(c + 1)

        wait(c)
        compute(c)
        return 0

    lax.fori_loop(0, _NCHUNKS, chunk_body, 0)

    pltpu.sync_copy(pos_s_v, pos_out_hbm.at[:, pl.ds(base, _BPW)])
    pltpu.sync_copy(neg_s_v, neg_out_hbm.at[:, pl.ds(base, _BPW)])


def _loss_body(pos_ref, neg_ref, pos_loss_ref, neg_loss_ref):
    p = pos_ref[...]
    lsp = jnp.minimum(p, 0.0) - jnp.log1p(jnp.exp(-jnp.abs(p)))
    pos_loss_ref[0, 0] = -jnp.sum(lsp) / float(BATCH)
    x = -neg_ref[...]
    lsn = jnp.minimum(x, 0.0) - jnp.log1p(jnp.exp(-jnp.abs(x)))
    neg_loss_ref[0, 0] = -jnp.sum(lsn) / float(BATCH * NEG)


_loss_call = pl.pallas_call(
    _loss_body,
    out_shape=(
        jax.ShapeDtypeStruct((1, 1), jnp.float32),
        jax.ShapeDtypeStruct((1, 1), jnp.float32),
    ),
    out_specs=(
        pl.BlockSpec(memory_space=pltpu.SMEM),
        pl.BlockSpec(memory_space=pltpu.SMEM),
    ),
)


def kernel(target_words, context_words, negative_words, input_emb, output_emb):
    tw = target_words.astype(jnp.int32)
    cw = context_words.astype(jnp.int32)
    nw = negative_words.astype(jnp.int32).T
    pos_scores, neg_scores = _sc_scores(tw, cw, nw, input_emb, output_emb)
    pos_loss, neg_loss = _loss_call(pos_scores, neg_scores)
    return (pos_loss[0, 0], neg_loss[0, 0])
